# baseline (speedup 1.0000x reference)
import jax
import jax.numpy as jnp
from jax import lax
from jax.experimental import pallas as pl
from jax.experimental.pallas import tpu as pltpu
from jax.experimental.pallas import tpu_sc as plsc

N = 10000
E = 160000
D = 256
DH = 128
N2 = 10240
NC = 2
NS = 16
NW = NC * NS
CH = 128
NCHP = 1280
CPT = NCHP // NS
CPW = NCHP // NW
ROWS_PER_TILE = N2 // NS

_mesh = plsc.VectorSubcoreMesh(
    core_axis_name="c", subcore_axis_name="s", num_cores=NC, num_subcores=NS
)
_sc_params = pltpu.CompilerParams(needs_layout_passes=False)


def _deg_body(src_hbm, dst_hbm, out_hbm, srcall, dstall, dego, degi):
    c = lax.axis_index("c")
    s = lax.axis_index("s")
    wid = s * NC + c
    zeros16 = jnp.zeros((16,), jnp.float32)
    ones16 = jnp.ones((16,), jnp.float32)

    pltpu.sync_copy(src_hbm.at[pl.ds(wid * CPW, CPW)], srcall)
    pltpu.sync_copy(dst_hbm.at[pl.ds(wid * CPW, CPW)], dstall)

    def zero_body(i, _):
        dego[pl.ds(i * 16, 16)] = zeros16
        degi[pl.ds(i * 16, 16)] = zeros16
        return 0

    lax.fori_loop(0, N2 // 16, zero_body, 0)

    def chunk_body(i, _):
        for j in range(CH // 16):
            si = srcall[i, pl.ds(16 * j, 16)]
            plsc.addupdate_scatter(dego, [si], ones16)
            di = dstall[i, pl.ds(16 * j, 16)]
            plsc.addupdate_scatter(degi, [di], ones16)
        return 0

    lax.fori_loop(0, CPW, chunk_body, 0)
    pltpu.sync_copy(dego, out_hbm.at[c, s, 0])
    pltpu.sync_copy(degi, out_hbm.at[c, s, 1])


_deg = pl.kernel(
    _deg_body,
    out_type=jax.ShapeDtypeStruct((NC, NS, 2, N2), jnp.float32),
    mesh=_mesh,
    scratch_types=[
        pltpu.VMEM((CPW, CH), jnp.int32),
        pltpu.VMEM((CPW, CH), jnp.int32),
        pltpu.VMEM((N2,), jnp.float32),
        pltpu.VMEM((N2,), jnp.float32),
    ],
    compiler_params=_sc_params,
)


def _agg_body(hs_hbm, src_hbm, dst_hbm, zrows_hbm, out_hbm,
              srcall, dstb0, dstb1, rows0, rows1, agg_sh,
              g0, g1, d0, d1):
    c = lax.axis_index("c")
    s = lax.axis_index("s")
    base = s * CPT

    pltpu.sync_copy(src_hbm.at[pl.ds(base, CPT)], srcall)

    def adj_body(k, _):
        i = k // (CH // 16)
        sl = pl.ds((k % (CH // 16)) * 16, 16)
        v = srcall[i, sl]
        srcall[i, sl] = v + v + c
        return 0

    lax.fori_loop(0, CPT * (CH // 16), adj_body, 0)

    pltpu.sync_copy(zrows_hbm, agg_sh.at[pl.ds(s * ROWS_PER_TILE,
                                               ROWS_PER_TILE)])
    plsc.subcore_barrier()

    def gather(ch, rows, sem):
        pltpu.async_copy(hs_hbm.at[srcall.at[ch]], rows, sem)

    def gwait(rows, sem):
        pltpu.make_async_copy(hs_hbm.at[srcall.at[0]], rows, sem).wait()

    def dstload(ch, buf, sem):
        pltpu.async_copy(dst_hbm.at[base + ch], buf, sem)

    def dwait(buf, sem):
        pltpu.make_async_copy(dst_hbm.at[base], buf, sem).wait()

    def scatter(rows, dbuf):
        pltpu.sync_copy(rows, agg_sh.at[dbuf], add=True)

    def chunk_body(i, _):
        dstload(i, dstb0, d0)
        gather(i, rows0, g0)
        gwait(rows0, g0)
        dwait(dstb0, d0)
        scatter(rows0, dstb0)
        return 0

    lax.fori_loop(0, CPT, chunk_body, 0)
    plsc.subcore_barrier()
    pltpu.sync_copy(
        agg_sh.at[pl.ds(s * ROWS_PER_TILE, ROWS_PER_TILE)],
        out_hbm.at[pl.ds(c * N2 + s * ROWS_PER_TILE, ROWS_PER_TILE)],
    )


_agg = pl.kernel(
    _agg_body,
    out_type=jax.ShapeDtypeStruct((NC * N2, DH), jnp.float32),
    mesh=_mesh,
    scratch_types=[
        pltpu.VMEM((CPT, CH), jnp.int32),
        pltpu.VMEM((CH,), jnp.int32),
        pltpu.VMEM((CH,), jnp.int32),
        pltpu.VMEM((CH, DH), jnp.float32),
        pltpu.VMEM((CH, DH), jnp.float32),
        pltpu.VMEM_SHARED((N2, DH), jnp.float32),
        pltpu.SemaphoreType.DMA,
        pltpu.SemaphoreType.DMA,
        pltpu.SemaphoreType.DMA,
        pltpu.SemaphoreType.DMA,
    ],
    compiler_params=_sc_params,
)


_HI = jax.lax.Precision.HIGHEST
_BN = 1024
_NB = N2 // _BN
_NORM_BN = 256


def _norms_body(degp_ref, ns_ref, nd_ref):
    d = jnp.sum(degp_ref[...], axis=0)
    bnl = d.shape[1]
    ii = lax.broadcasted_iota(jnp.int32, (bnl, bnl), 0)
    jj = lax.broadcasted_iota(jnp.int32, (bnl, bnl), 1)
    ident = jnp.where(ii == jj, 1.0, 0.0)
    cols = lax.dot_general(ident, d, (((1,), (1,)), ((), ())), precision=_HI)
    deg_out = cols[:, 0:1]
    deg_in = cols[:, 1:2]
    ns_ref[...] = jnp.where(deg_out > 0.0,
                            lax.rsqrt(jnp.maximum(deg_out, 1e-12)), 0.0)
    nd_ref[...] = jnp.where(deg_in > 0.0,
                            lax.rsqrt(jnp.maximum(deg_in, 1e-12)), 0.0)


def _norms(degp):
    return pl.pallas_call(
        _norms_body,
        grid=(N2 // _NORM_BN,),
        in_specs=[pl.BlockSpec((NW, 2, _NORM_BN), lambda b: (0, 0, b))],
        out_specs=[
            pl.BlockSpec((_NORM_BN, 1), lambda b: (b, 0)),
            pl.BlockSpec((_NORM_BN, 1), lambda b: (b, 0)),
        ],
        out_shape=[
            jax.ShapeDtypeStruct((N2, 1), jnp.float32),
            jax.ShapeDtypeStruct((N2, 1), jnp.float32),
        ],
    )(degp)


def _mm1_body(x_ref, w_ref, ns_ref, out_ref):
    h = jnp.dot(x_ref[...], w_ref[...], precision=_HI)
    hs = h * ns_ref[...]
    out_ref[...] = hs.reshape(_BN, NC, DH)


def _mm1(xp, W1, ns):
    return pl.pallas_call(
        _mm1_body,
        grid=(_NB,),
        in_specs=[
            pl.BlockSpec((_BN, D), lambda b: (b, 0)),
            pl.BlockSpec((D, D), lambda b: (0, 0)),
            pl.BlockSpec((_BN, 1), lambda b: (b, 0)),
        ],
        out_specs=pl.BlockSpec((_BN, NC, DH), lambda b: (b, 0, 0)),
        out_shape=jax.ShapeDtypeStruct((N2, NC, DH), jnp.float32),
    )(xp, W1, ns)


def _mm2_body(a0_ref, a1_ref, nd_ref, b1_ref, w_ref, ns_ref, out_ref):
    a = jnp.concatenate([a0_ref[...], a1_ref[...]], axis=1)
    t = jnp.maximum(a * nd_ref[...] + b1_ref[...], 0.0)
    h = jnp.dot(t, w_ref[...], precision=_HI)
    hs = h * ns_ref[...]
    out_ref[...] = hs.reshape(_BN, NC, DH)


def _mm2(agg1, nd, b1, W2, ns):
    return pl.pallas_call(
        _mm2_body,
        grid=(_NB,),
        in_specs=[
            pl.BlockSpec((_BN, DH), lambda b: (b, 0)),
            pl.BlockSpec((_BN, DH), lambda b: (b + _NB, 0)),
            pl.BlockSpec((_BN, 1), lambda b: (b, 0)),
            pl.BlockSpec((1, D), lambda b: (0, 0)),
            pl.BlockSpec((D, D), lambda b: (0, 0)),
            pl.BlockSpec((_BN, 1), lambda b: (b, 0)),
        ],
        out_specs=pl.BlockSpec((_BN, NC, DH), lambda b: (b, 0, 0)),
        out_shape=jax.ShapeDtypeStruct((N2, NC, DH), jnp.float32),
    )(agg1, agg1, nd, b1, W2, ns)


def _final_body(a0_ref, a1_ref, nd_ref, b2_ref, out_ref):
    a = jnp.concatenate([a0_ref[...], a1_ref[...]], axis=1)
    out_ref[...] = a * nd_ref[...] + b2_ref[...]


def _final(agg2, nd, b2):
    return pl.pallas_call(
        _final_body,
        grid=(_NB,),
        in_specs=[
            pl.BlockSpec((_BN, DH), lambda b: (b, 0)),
            pl.BlockSpec((_BN, DH), lambda b: (b + _NB, 0)),
            pl.BlockSpec((_BN, 1), lambda b: (b, 0)),
            pl.BlockSpec((1, D), lambda b: (0, 0)),
        ],
        out_specs=pl.BlockSpec((_BN, D), lambda b: (b, 0)),
        out_shape=jax.ShapeDtypeStruct((N2, D), jnp.float32),
    )(agg2, agg2, nd, b2)


@jax.jit
def kernel(x, edge_index, W1, b1, W2, b2):
    pad = jnp.full((NCHP * CH - E,), N, jnp.int32)
    srcd = jnp.concatenate([edge_index[0], pad]).reshape(NCHP, CH)
    dstd = jnp.concatenate([edge_index[1], pad]).reshape(NCHP, CH)

    degp = _deg(srcd, dstd).reshape(NC * NS, 2, N2)
    ns, nd = _norms(degp)

    xp = jnp.pad(x, ((0, N2 - N), (0, 0)))
    zrows = jnp.zeros((ROWS_PER_TILE, DH), jnp.float32)

    hs1 = _mm1(xp, W1, ns).reshape(NC * N2, DH)
    agg1 = _agg(hs1, srcd, dstd, zrows)

    hs2 = _mm2(agg1, nd, b1.reshape(1, D), W2, ns).reshape(NC * N2, DH)
    agg2 = _agg(hs2, srcd, dstd, zrows)

    out = _final(agg2, nd, b2.reshape(1, D))
    return out[:N]


# speedup vs baseline: 1.2575x; 1.2575x over previous
import jax
import jax.numpy as jnp
from jax import lax
from jax.experimental import pallas as pl
from jax.experimental.pallas import tpu as pltpu
from jax.experimental.pallas import tpu_sc as plsc

N = 10000
E = 160000
D = 256
DH = 128
N2 = 10240
NC = 2
NS = 16
NW = NC * NS
CH = 128
NCHP = 1280
CPT = NCHP // NS
CPW = NCHP // NW
ROWS_PER_TILE = N2 // NS

_mesh = plsc.VectorSubcoreMesh(
    core_axis_name="c", subcore_axis_name="s", num_cores=NC, num_subcores=NS
)
_sc_params = pltpu.CompilerParams(needs_layout_passes=False)


def _deg_body(src_hbm, dst_hbm, out_hbm, srcall, dstall, dego, degi):
    c = lax.axis_index("c")
    s = lax.axis_index("s")
    wid = s * NC + c
    zeros16 = jnp.zeros((16,), jnp.float32)
    ones16 = jnp.ones((16,), jnp.float32)

    pltpu.sync_copy(src_hbm.at[pl.ds(wid * CPW, CPW)], srcall)
    pltpu.sync_copy(dst_hbm.at[pl.ds(wid * CPW, CPW)], dstall)

    def zero_body(i, _):
        dego[pl.ds(i * 16, 16)] = zeros16
        degi[pl.ds(i * 16, 16)] = zeros16
        return 0

    lax.fori_loop(0, N2 // 16, zero_body, 0)

    def chunk_body(i, _):
        for j in range(CH // 16):
            si = srcall[i, pl.ds(16 * j, 16)]
            plsc.addupdate_scatter(dego, [si], ones16)
            di = dstall[i, pl.ds(16 * j, 16)]
            plsc.addupdate_scatter(degi, [di], ones16)
        return 0

    lax.fori_loop(0, CPW, chunk_body, 0)
    pltpu.sync_copy(dego, out_hbm.at[c, s, 0])
    pltpu.sync_copy(degi, out_hbm.at[c, s, 1])


_deg = pl.kernel(
    _deg_body,
    out_type=jax.ShapeDtypeStruct((NC, NS, 2, N2), jnp.float32),
    mesh=_mesh,
    scratch_types=[
        pltpu.VMEM((CPW, CH), jnp.int32),
        pltpu.VMEM((CPW, CH), jnp.int32),
        pltpu.VMEM((N2,), jnp.float32),
        pltpu.VMEM((N2,), jnp.float32),
    ],
    compiler_params=_sc_params,
)


def _agg_body(hs_hbm, src_hbm, dst_hbm, zrows_hbm, out_hbm,
              srcall, dstb0, dstb1, rows0, rows1, agg_sh,
              g0, g1, d0, d1):
    c = lax.axis_index("c")
    s = lax.axis_index("s")
    base = s * CPT

    pltpu.sync_copy(src_hbm.at[pl.ds(base, CPT)], srcall)

    off = c * N2

    def adj_body(k, _):
        i = k // (CH // 16)
        sl = pl.ds((k % (CH // 16)) * 16, 16)
        srcall[i, sl] = srcall[i, sl] + off
        return 0

    lax.fori_loop(0, CPT * (CH // 16), adj_body, 0)

    pltpu.sync_copy(zrows_hbm, agg_sh.at[pl.ds(s * ROWS_PER_TILE,
                                               ROWS_PER_TILE)])
    plsc.subcore_barrier()

    def gather(ch, rows, sem):
        pltpu.async_copy(hs_hbm.at[srcall.at[ch]], rows, sem)

    def gwait(rows, sem):
        pltpu.make_async_copy(hs_hbm.at[srcall.at[0]], rows, sem).wait()

    def dstload(ch, buf, sem):
        pltpu.async_copy(dst_hbm.at[base + ch], buf, sem)

    def dwait(buf, sem):
        pltpu.make_async_copy(dst_hbm.at[base], buf, sem).wait()

    def scatter(rows, dbuf):
        pltpu.sync_copy(rows, agg_sh.at[dbuf], add=True)

    dstload(0, dstb0, d0)
    gather(0, rows0, g0)
    dstload(1, dstb1, d1)
    last = CPT // 2 - 1

    def pair_body(i2, _):
        ca = 2 * i2
        gwait(rows0, g0)
        gather(ca + 1, rows1, g1)
        dwait(dstb0, d0)
        scatter(rows0, dstb0)

        @pl.when(i2 < last)
        def _():
            dstload(ca + 2, dstb0, d0)
            gather(ca + 2, rows0, g0)

        gwait(rows1, g1)
        dwait(dstb1, d1)
        scatter(rows1, dstb1)

        @pl.when(i2 < last)
        def _():
            dstload(ca + 3, dstb1, d1)

        return 0

    lax.fori_loop(0, CPT // 2, pair_body, 0)
    plsc.subcore_barrier()
    pltpu.sync_copy(
        agg_sh.at[pl.ds(s * ROWS_PER_TILE, ROWS_PER_TILE)],
        out_hbm.at[pl.ds(c * N2 + s * ROWS_PER_TILE, ROWS_PER_TILE)],
    )


_agg = pl.kernel(
    _agg_body,
    out_type=jax.ShapeDtypeStruct((NC * N2, DH), jnp.float32),
    mesh=_mesh,
    scratch_types=[
        pltpu.VMEM((CPT, CH), jnp.int32),
        pltpu.VMEM((CH,), jnp.int32),
        pltpu.VMEM((CH,), jnp.int32),
        pltpu.VMEM((CH, DH), jnp.float32),
        pltpu.VMEM((CH, DH), jnp.float32),
        pltpu.VMEM_SHARED((N2, DH), jnp.float32),
        pltpu.SemaphoreType.DMA,
        pltpu.SemaphoreType.DMA,
        pltpu.SemaphoreType.DMA,
        pltpu.SemaphoreType.DMA,
    ],
    compiler_params=_sc_params,
)


_HI = jax.lax.Precision.HIGHEST
_BN = 1024
_NB = N2 // _BN
_NORM_BN = 256


def _norms_body(degp_ref, ns_ref, nd_ref):
    d = jnp.sum(degp_ref[...], axis=0)
    bnl = d.shape[1]
    ii = lax.broadcasted_iota(jnp.int32, (bnl, bnl), 0)
    jj = lax.broadcasted_iota(jnp.int32, (bnl, bnl), 1)
    ident = jnp.where(ii == jj, 1.0, 0.0)
    cols = lax.dot_general(ident, d, (((1,), (1,)), ((), ())), precision=_HI)
    deg_out = cols[:, 0:1]
    deg_in = cols[:, 1:2]
    ns_ref[...] = jnp.where(deg_out > 0.0,
                            lax.rsqrt(jnp.maximum(deg_out, 1e-12)), 0.0)
    nd_ref[...] = jnp.where(deg_in > 0.0,
                            lax.rsqrt(jnp.maximum(deg_in, 1e-12)), 0.0)


def _norms(degp):
    return pl.pallas_call(
        _norms_body,
        grid=(N2 // _NORM_BN,),
        in_specs=[pl.BlockSpec((NW, 2, _NORM_BN), lambda b: (0, 0, b))],
        out_specs=[
            pl.BlockSpec((_NORM_BN, 1), lambda b: (b, 0)),
            pl.BlockSpec((_NORM_BN, 1), lambda b: (b, 0)),
        ],
        out_shape=[
            jax.ShapeDtypeStruct((N2, 1), jnp.float32),
            jax.ShapeDtypeStruct((N2, 1), jnp.float32),
        ],
    )(degp)


def _mm1_body(x_ref, w_ref, ns_ref, p0_ref, p1_ref):
    h = jnp.dot(x_ref[...], w_ref[...], precision=_HI)
    hs = h * ns_ref[...]
    p0_ref[...] = hs[:, :DH]
    p1_ref[...] = hs[:, DH:]


def _mm1(xp, W1, ns):
    return pl.pallas_call(
        _mm1_body,
        grid=(_NB,),
        in_specs=[
            pl.BlockSpec((_BN, D), lambda b: (b, 0)),
            pl.BlockSpec((D, D), lambda b: (0, 0)),
            pl.BlockSpec((_BN, 1), lambda b: (b, 0)),
        ],
        out_specs=[
            pl.BlockSpec((_BN, DH), lambda b: (b, 0)),
            pl.BlockSpec((_BN, DH), lambda b: (b, 0)),
        ],
        out_shape=[
            jax.ShapeDtypeStruct((N2, DH), jnp.float32),
            jax.ShapeDtypeStruct((N2, DH), jnp.float32),
        ],
    )(xp, W1, ns)


def _mm2_body(a0_ref, a1_ref, nd_ref, b1_ref, w_ref, ns_ref, p0_ref, p1_ref):
    a = jnp.concatenate([a0_ref[...], a1_ref[...]], axis=1)
    t = jnp.maximum(a * nd_ref[...] + b1_ref[...], 0.0)
    h = jnp.dot(t, w_ref[...], precision=_HI)
    hs = h * ns_ref[...]
    p0_ref[...] = hs[:, :DH]
    p1_ref[...] = hs[:, DH:]


def _mm2(agg1, nd, b1, W2, ns):
    return pl.pallas_call(
        _mm2_body,
        grid=(_NB,),
        in_specs=[
            pl.BlockSpec((_BN, DH), lambda b: (b, 0)),
            pl.BlockSpec((_BN, DH), lambda b: (b + _NB, 0)),
            pl.BlockSpec((_BN, 1), lambda b: (b, 0)),
            pl.BlockSpec((1, D), lambda b: (0, 0)),
            pl.BlockSpec((D, D), lambda b: (0, 0)),
            pl.BlockSpec((_BN, 1), lambda b: (b, 0)),
        ],
        out_specs=[
            pl.BlockSpec((_BN, DH), lambda b: (b, 0)),
            pl.BlockSpec((_BN, DH), lambda b: (b, 0)),
        ],
        out_shape=[
            jax.ShapeDtypeStruct((N2, DH), jnp.float32),
            jax.ShapeDtypeStruct((N2, DH), jnp.float32),
        ],
    )(agg1, agg1, nd, b1, W2, ns)


def _final_body(a0_ref, a1_ref, nd_ref, b2_ref, out_ref):
    a = jnp.concatenate([a0_ref[...], a1_ref[...]], axis=1)
    out_ref[...] = a * nd_ref[...] + b2_ref[...]


def _final(agg2, nd, b2):
    return pl.pallas_call(
        _final_body,
        grid=(_NB,),
        in_specs=[
            pl.BlockSpec((_BN, DH), lambda b: (b, 0)),
            pl.BlockSpec((_BN, DH), lambda b: (b + _NB, 0)),
            pl.BlockSpec((_BN, 1), lambda b: (b, 0)),
            pl.BlockSpec((1, D), lambda b: (0, 0)),
        ],
        out_specs=pl.BlockSpec((_BN, D), lambda b: (b, 0)),
        out_shape=jax.ShapeDtypeStruct((N2, D), jnp.float32),
    )(agg2, agg2, nd, b2)


@jax.jit
def kernel(x, edge_index, W1, b1, W2, b2):
    pad = jnp.full((NCHP * CH - E,), N, jnp.int32)
    srcd = jnp.concatenate([edge_index[0], pad]).reshape(NCHP, CH)
    dstd = jnp.concatenate([edge_index[1], pad]).reshape(NCHP, CH)

    degp = _deg(srcd, dstd).reshape(NC * NS, 2, N2)
    ns, nd = _norms(degp)

    xp = jnp.pad(x, ((0, N2 - N), (0, 0)))
    zrows = jnp.zeros((ROWS_PER_TILE, DH), jnp.float32)

    p0, p1 = _mm1(xp, W1, ns)
    hs1 = jnp.concatenate([p0, p1], axis=0)
    agg1 = _agg(hs1, srcd, dstd, zrows)

    p0, p1 = _mm2(agg1, nd, b1.reshape(1, D), W2, ns)
    hs2 = jnp.concatenate([p0, p1], axis=0)
    agg2 = _agg(hs2, srcd, dstd, zrows)

    out = _final(agg2, nd, b2.reshape(1, D))
    return out[:N]


# speedup vs baseline: 1.2950x; 1.0298x over previous
import jax
import jax.numpy as jnp
from jax import lax
from jax.experimental import pallas as pl
from jax.experimental.pallas import tpu as pltpu
from jax.experimental.pallas import tpu_sc as plsc

N = 10000
E = 160000
D = 256
DH = 128
N2 = 10240
NC = 2
NS = 16
NW = NC * NS
CH = 128
NCHP = 1280
CPT = NCHP // NS
CPW = NCHP // NW
ROWS_PER_TILE = N2 // NS

_mesh = plsc.VectorSubcoreMesh(
    core_axis_name="c", subcore_axis_name="s", num_cores=NC, num_subcores=NS
)
_sc_params = pltpu.CompilerParams(needs_layout_passes=False)


def _deg_body(src_hbm, dst_hbm, out_hbm, srcall, dstall, dego, degi):
    c = lax.axis_index("c")
    s = lax.axis_index("s")
    wid = s * NC + c
    zeros16 = jnp.zeros((16,), jnp.float32)
    ones16 = jnp.ones((16,), jnp.float32)

    pltpu.sync_copy(src_hbm.at[pl.ds(wid * CPW, CPW)], srcall)
    pltpu.sync_copy(dst_hbm.at[pl.ds(wid * CPW, CPW)], dstall)

    def zero_body(i, _):
        dego[pl.ds(i * 16, 16)] = zeros16
        degi[pl.ds(i * 16, 16)] = zeros16
        return 0

    lax.fori_loop(0, N2 // 16, zero_body, 0)

    def chunk_body(i, _):
        for j in range(CH // 16):
            si = srcall[i, pl.ds(16 * j, 16)]
            plsc.addupdate_scatter(dego, [si], ones16)
            di = dstall[i, pl.ds(16 * j, 16)]
            plsc.addupdate_scatter(degi, [di], ones16)
        return 0

    lax.fori_loop(0, CPW, chunk_body, 0)
    pltpu.sync_copy(dego, out_hbm.at[c, s, 0])
    pltpu.sync_copy(degi, out_hbm.at[c, s, 1])


_deg = pl.kernel(
    _deg_body,
    out_type=jax.ShapeDtypeStruct((NC, NS, 2, N2), jnp.float32),
    mesh=_mesh,
    scratch_types=[
        pltpu.VMEM((CPW, CH), jnp.int32),
        pltpu.VMEM((CPW, CH), jnp.int32),
        pltpu.VMEM((N2,), jnp.float32),
        pltpu.VMEM((N2,), jnp.float32),
    ],
    compiler_params=_sc_params,
)


def _agg_body(hs_hbm, src_hbm, dst_hbm, zrows_hbm, out_hbm,
              srcb0, srcb1, dstb0, dstb1, rows0, rows1, agg_sh,
              s0, s1, g0, g1, d0, d1):
    c = lax.axis_index("c")
    s = lax.axis_index("s")
    base = s * CPT
    off = c * N2

    pltpu.sync_copy(zrows_hbm, agg_sh.at[pl.ds(s * ROWS_PER_TILE,
                                               ROWS_PER_TILE)])
    plsc.subcore_barrier()

    def srcload(ch, buf, sem):
        pltpu.async_copy(src_hbm.at[base + ch], buf, sem)

    def dstload(ch, buf, sem):
        pltpu.async_copy(dst_hbm.at[base + ch], buf, sem)

    def iwait(buf, sem):
        pltpu.make_async_copy(src_hbm.at[base], buf, sem).wait()

    def adjust(buf):
        for j in range(CH // 16):
            sl = pl.ds(16 * j, 16)
            buf[sl] = buf[sl] + off

    def gather(rows, buf, sem):
        pltpu.async_copy(hs_hbm.at[buf], rows, sem)

    def gwait(rows, buf, sem):
        pltpu.make_async_copy(hs_hbm.at[buf], rows, sem).wait()

    def scatter(rows, dbuf):
        pltpu.sync_copy(rows, agg_sh.at[dbuf], add=True)

    srcload(0, srcb0, s0)
    dstload(0, dstb0, d0)
    srcload(1, srcb1, s1)
    dstload(1, dstb1, d1)
    iwait(srcb0, s0)
    adjust(srcb0)
    gather(rows0, srcb0, g0)
    last = CPT // 2 - 1

    def pair_body(i2, _):
        ca = 2 * i2
        iwait(srcb1, s1)
        adjust(srcb1)
        gather(rows1, srcb1, g1)
        gwait(rows0, srcb0, g0)

        @pl.when(i2 < last)
        def _():
            srcload(ca + 2, srcb0, s0)

        pltpu.make_async_copy(dst_hbm.at[base], dstb0, d0).wait()
        scatter(rows0, dstb0)

        @pl.when(i2 < last)
        def _():
            dstload(ca + 2, dstb0, d0)
            iwait(srcb0, s0)
            adjust(srcb0)
            gather(rows0, srcb0, g0)

        gwait(rows1, srcb1, g1)

        @pl.when(i2 < last)
        def _():
            srcload(ca + 3, srcb1, s1)

        pltpu.make_async_copy(dst_hbm.at[base], dstb1, d1).wait()
        scatter(rows1, dstb1)

        @pl.when(i2 < last)
        def _():
            dstload(ca + 3, dstb1, d1)

        return 0

    lax.fori_loop(0, CPT // 2, pair_body, 0)
    plsc.subcore_barrier()
    pltpu.sync_copy(
        agg_sh.at[pl.ds(s * ROWS_PER_TILE, ROWS_PER_TILE)],
        out_hbm.at[pl.ds(c * N2 + s * ROWS_PER_TILE, ROWS_PER_TILE)],
    )


_agg = pl.kernel(
    _agg_body,
    out_type=jax.ShapeDtypeStruct((NC * N2, DH), jnp.float32),
    mesh=_mesh,
    scratch_types=[
        pltpu.VMEM((CH,), jnp.int32),
        pltpu.VMEM((CH,), jnp.int32),
        pltpu.VMEM((CH,), jnp.int32),
        pltpu.VMEM((CH,), jnp.int32),
        pltpu.VMEM((CH, DH), jnp.float32),
        pltpu.VMEM((CH, DH), jnp.float32),
        pltpu.VMEM_SHARED((N2, DH), jnp.float32),
        pltpu.SemaphoreType.DMA,
        pltpu.SemaphoreType.DMA,
        pltpu.SemaphoreType.DMA,
        pltpu.SemaphoreType.DMA,
        pltpu.SemaphoreType.DMA,
        pltpu.SemaphoreType.DMA,
    ],
    compiler_params=_sc_params,
)


_HI = jax.lax.Precision.HIGHEST
_BN = 1024
_NB = N2 // _BN
_NORM_BN = 256


def _norms_body(degp_ref, ns_ref, nd_ref):
    d = jnp.sum(degp_ref[...], axis=0)
    bnl = d.shape[1]
    ii = lax.broadcasted_iota(jnp.int32, (bnl, bnl), 0)
    jj = lax.broadcasted_iota(jnp.int32, (bnl, bnl), 1)
    ident = jnp.where(ii == jj, 1.0, 0.0)
    cols = lax.dot_general(ident, d, (((1,), (1,)), ((), ())), precision=_HI)
    deg_out = cols[:, 0:1]
    deg_in = cols[:, 1:2]
    ns_ref[...] = jnp.where(deg_out > 0.0,
                            lax.rsqrt(jnp.maximum(deg_out, 1e-12)), 0.0)
    nd_ref[...] = jnp.where(deg_in > 0.0,
                            lax.rsqrt(jnp.maximum(deg_in, 1e-12)), 0.0)


def _norms(degp):
    return pl.pallas_call(
        _norms_body,
        grid=(N2 // _NORM_BN,),
        in_specs=[pl.BlockSpec((NW, 2, _NORM_BN), lambda b: (0, 0, b))],
        out_specs=[
            pl.BlockSpec((_NORM_BN, 1), lambda b: (b, 0)),
            pl.BlockSpec((_NORM_BN, 1), lambda b: (b, 0)),
        ],
        out_shape=[
            jax.ShapeDtypeStruct((N2, 1), jnp.float32),
            jax.ShapeDtypeStruct((N2, 1), jnp.float32),
        ],
    )(degp)


def _mm1_body(x_ref, w_ref, ns_ref, p0_ref, p1_ref):
    h = jnp.dot(x_ref[...], w_ref[...], precision=_HI)
    hs = h * ns_ref[...]
    p0_ref[...] = hs[:, :DH]
    p1_ref[...] = hs[:, DH:]


def _mm1(xp, W1, ns):
    return pl.pallas_call(
        _mm1_body,
        grid=(_NB,),
        in_specs=[
            pl.BlockSpec((_BN, D), lambda b: (b, 0)),
            pl.BlockSpec((D, D), lambda b: (0, 0)),
            pl.BlockSpec((_BN, 1), lambda b: (b, 0)),
        ],
        out_specs=[
            pl.BlockSpec((_BN, DH), lambda b: (b, 0)),
            pl.BlockSpec((_BN, DH), lambda b: (b, 0)),
        ],
        out_shape=[
            jax.ShapeDtypeStruct((N2, DH), jnp.float32),
            jax.ShapeDtypeStruct((N2, DH), jnp.float32),
        ],
    )(xp, W1, ns)


def _mm2_body(a0_ref, a1_ref, nd_ref, b1_ref, w_ref, ns_ref, p0_ref, p1_ref):
    a = jnp.concatenate([a0_ref[...], a1_ref[...]], axis=1)
    t = jnp.maximum(a * nd_ref[...] + b1_ref[...], 0.0)
    h = jnp.dot(t, w_ref[...], precision=_HI)
    hs = h * ns_ref[...]
    p0_ref[...] = hs[:, :DH]
    p1_ref[...] = hs[:, DH:]


def _mm2(agg1, nd, b1, W2, ns):
    return pl.pallas_call(
        _mm2_body,
        grid=(_NB,),
        in_specs=[
            pl.BlockSpec((_BN, DH), lambda b: (b, 0)),
            pl.BlockSpec((_BN, DH), lambda b: (b + _NB, 0)),
            pl.BlockSpec((_BN, 1), lambda b: (b, 0)),
            pl.BlockSpec((1, D), lambda b: (0, 0)),
            pl.BlockSpec((D, D), lambda b: (0, 0)),
            pl.BlockSpec((_BN, 1), lambda b: (b, 0)),
        ],
        out_specs=[
            pl.BlockSpec((_BN, DH), lambda b: (b, 0)),
            pl.BlockSpec((_BN, DH), lambda b: (b, 0)),
        ],
        out_shape=[
            jax.ShapeDtypeStruct((N2, DH), jnp.float32),
            jax.ShapeDtypeStruct((N2, DH), jnp.float32),
        ],
    )(agg1, agg1, nd, b1, W2, ns)


def _final_body(a0_ref, a1_ref, nd_ref, b2_ref, out_ref):
    a = jnp.concatenate([a0_ref[...], a1_ref[...]], axis=1)
    out_ref[...] = a * nd_ref[...] + b2_ref[...]


def _final(agg2, nd, b2):
    return pl.pallas_call(
        _final_body,
        grid=(_NB,),
        in_specs=[
            pl.BlockSpec((_BN, DH), lambda b: (b, 0)),
            pl.BlockSpec((_BN, DH), lambda b: (b + _NB, 0)),
            pl.BlockSpec((_BN, 1), lambda b: (b, 0)),
            pl.BlockSpec((1, D), lambda b: (0, 0)),
        ],
        out_specs=pl.BlockSpec((_BN, D), lambda b: (b, 0)),
        out_shape=jax.ShapeDtypeStruct((N2, D), jnp.float32),
    )(agg2, agg2, nd, b2)


@jax.jit
def kernel(x, edge_index, W1, b1, W2, b2):
    pad = jnp.full((NCHP * CH - E,), N, jnp.int32)
    srcd = jnp.concatenate([edge_index[0], pad]).reshape(NCHP, CH)
    dstd = jnp.concatenate([edge_index[1], pad]).reshape(NCHP, CH)

    degp = _deg(srcd, dstd).reshape(NC * NS, 2, N2)
    ns, nd = _norms(degp)

    xp = jnp.pad(x, ((0, N2 - N), (0, 0)))
    zrows = jnp.zeros((ROWS_PER_TILE, DH), jnp.float32)

    p0, p1 = _mm1(xp, W1, ns)
    hs1 = jnp.concatenate([p0, p1], axis=0)
    agg1 = _agg(hs1, srcd, dstd, zrows)

    p0, p1 = _mm2(agg1, nd, b1.reshape(1, D), W2, ns)
    hs2 = jnp.concatenate([p0, p1], axis=0)
    agg2 = _agg(hs2, srcd, dstd, zrows)

    out = _final(agg2, nd, b2.reshape(1, D))
    return out[:N]


# speedup vs baseline: 1.4213x; 1.0975x over previous
import jax
import jax.numpy as jnp
from jax import lax
from jax.experimental import pallas as pl
from jax.experimental.pallas import tpu as pltpu
from jax.experimental.pallas import tpu_sc as plsc

N = 10000
E = 160000
D = 256
DH = 128
N2 = 10240
NC = 2
NS = 16
NW = NC * NS
CH = 128
NCHUNK = E // CH
NCHP = 1280
CPT = NCHP // NS
CPW = NCHP // NW
ROWS_PER_TILE = N2 // NS

_mesh = plsc.VectorSubcoreMesh(
    core_axis_name="c", subcore_axis_name="s", num_cores=NC, num_subcores=NS
)
_sc_params = pltpu.CompilerParams(needs_layout_passes=False)


def _deg_body(src_hbm, dst_hbm, out_hbm, srcall, dstall, dego, degi):
    c = lax.axis_index("c")
    s = lax.axis_index("s")
    wid = s * NC + c
    zeros16 = jnp.zeros((16,), jnp.float32)
    ones16 = jnp.ones((16,), jnp.float32)

    pltpu.sync_copy(src_hbm.at[pl.ds(wid * CPW, CPW)], srcall)
    pltpu.sync_copy(dst_hbm.at[pl.ds(wid * CPW, CPW)], dstall)

    def zero_body(i, _):
        dego[pl.ds(i * 16, 16)] = zeros16
        degi[pl.ds(i * 16, 16)] = zeros16
        return 0

    lax.fori_loop(0, N2 // 16, zero_body, 0)

    def chunk_body(i, _):
        for j in range(CH // 16):
            si = srcall[i, pl.ds(16 * j, 16)]
            plsc.addupdate_scatter(dego, [si], ones16)
            di = dstall[i, pl.ds(16 * j, 16)]
            plsc.addupdate_scatter(degi, [di], ones16)
        return 0

    lax.fori_loop(0, CPW, chunk_body, 0)
    pltpu.sync_copy(dego, out_hbm.at[c, s, 0])
    pltpu.sync_copy(degi, out_hbm.at[c, s, 1])


_deg = pl.kernel(
    _deg_body,
    out_type=jax.ShapeDtypeStruct((NC, NS, 2, N2), jnp.float32),
    mesh=_mesh,
    scratch_types=[
        pltpu.VMEM((CPW, CH), jnp.int32),
        pltpu.VMEM((CPW, CH), jnp.int32),
        pltpu.VMEM((N2,), jnp.float32),
        pltpu.VMEM((N2,), jnp.float32),
    ],
    compiler_params=_sc_params,
)


def _agg_body(hs_hbm, src_hbm, dst_hbm, zrows_hbm, out_hbm,
              srcb0, dstb0, rows0, agg_sh, g0):
    c = lax.axis_index("c")
    s = lax.axis_index("s")
    off = c * N2

    pltpu.sync_copy(zrows_hbm, agg_sh.at[pl.ds(s * ROWS_PER_TILE,
                                               ROWS_PER_TILE)])
    plsc.subcore_barrier()

    def adjust(buf):
        for j in range(CH // 16):
            sl = pl.ds(16 * j, 16)
            buf[sl] = buf[sl] + off

    nch = NCHUNK // NS + jnp.where(s < NCHUNK - (NCHUNK // NS) * NS, 1, 0)
    cbase = s * (NCHUNK // NS) + jnp.minimum(s, NCHUNK - (NCHUNK // NS) * NS)

    def chunk_body(i, _):
        ch = cbase + i
        pltpu.sync_copy(src_hbm.at[ch], srcb0)
        pltpu.sync_copy(dst_hbm.at[ch], dstb0)
        adjust(srcb0)
        pltpu.async_copy(hs_hbm.at[srcb0], rows0, g0).wait()
        pltpu.sync_copy(rows0, agg_sh.at[dstb0], add=True)
        return 0

    lax.fori_loop(0, nch, chunk_body, 0)
    plsc.subcore_barrier()
    pltpu.sync_copy(
        agg_sh.at[pl.ds(s * ROWS_PER_TILE, ROWS_PER_TILE)],
        out_hbm.at[pl.ds(c * N2 + s * ROWS_PER_TILE, ROWS_PER_TILE)],
    )


_agg = pl.kernel(
    _agg_body,
    out_type=jax.ShapeDtypeStruct((NC * N2, DH), jnp.float32),
    mesh=_mesh,
    scratch_types=[
        pltpu.VMEM((CH,), jnp.int32),
        pltpu.VMEM((CH,), jnp.int32),
        pltpu.VMEM((CH, DH), jnp.float32),
        pltpu.VMEM_SHARED((N2, DH), jnp.float32),
        pltpu.SemaphoreType.DMA,
    ],
    compiler_params=_sc_params,
)


_HI = jax.lax.Precision.HIGHEST
_BN = 1024
_NB = N2 // _BN
_NORM_BN = 256


def _norms_body(degp_ref, ns_ref, nd_ref):
    d = jnp.sum(degp_ref[...], axis=0)
    bnl = d.shape[1]
    ii = lax.broadcasted_iota(jnp.int32, (bnl, bnl), 0)
    jj = lax.broadcasted_iota(jnp.int32, (bnl, bnl), 1)
    ident = jnp.where(ii == jj, 1.0, 0.0)
    cols = lax.dot_general(ident, d, (((1,), (1,)), ((), ())), precision=_HI)
    deg_out = cols[:, 0:1]
    deg_in = cols[:, 1:2]
    ns_ref[...] = jnp.where(deg_out > 0.0,
                            lax.rsqrt(jnp.maximum(deg_out, 1e-12)), 0.0)
    nd_ref[...] = jnp.where(deg_in > 0.0,
                            lax.rsqrt(jnp.maximum(deg_in, 1e-12)), 0.0)


def _norms(degp):
    return pl.pallas_call(
        _norms_body,
        grid=(N2 // _NORM_BN,),
        in_specs=[pl.BlockSpec((NW, 2, _NORM_BN), lambda b: (0, 0, b))],
        out_specs=[
            pl.BlockSpec((_NORM_BN, 1), lambda b: (b, 0)),
            pl.BlockSpec((_NORM_BN, 1), lambda b: (b, 0)),
        ],
        out_shape=[
            jax.ShapeDtypeStruct((N2, 1), jnp.float32),
            jax.ShapeDtypeStruct((N2, 1), jnp.float32),
        ],
    )(degp)


def _mm1_body(x_ref, w_ref, ns_ref, out_ref):
    p = pl.program_id(1)
    h = jnp.dot(x_ref[...], w_ref[...], precision=_HI)
    hs = h * ns_ref[...]
    out_ref[...] = jnp.where(p == 0, hs[:, :DH], hs[:, DH:])


def _mm1(xp, W1, ns):
    return pl.pallas_call(
        _mm1_body,
        grid=(_NB, NC),
        in_specs=[
            pl.BlockSpec((_BN, D), lambda b, p: (b, 0)),
            pl.BlockSpec((D, D), lambda b, p: (0, 0)),
            pl.BlockSpec((_BN, 1), lambda b, p: (b, 0)),
        ],
        out_specs=pl.BlockSpec((_BN, DH), lambda b, p: (p * _NB + b, 0)),
        out_shape=jax.ShapeDtypeStruct((NC * N2, DH), jnp.float32),
    )(xp, W1, ns)


def _mm2_body(a0_ref, a1_ref, nd_ref, b1_ref, w_ref, ns_ref, out_ref):
    p = pl.program_id(1)
    a = jnp.concatenate([a0_ref[...], a1_ref[...]], axis=1)
    t = jnp.maximum(a * nd_ref[...] + b1_ref[...], 0.0)
    h = jnp.dot(t, w_ref[...], precision=_HI)
    hs = h * ns_ref[...]
    out_ref[...] = jnp.where(p == 0, hs[:, :DH], hs[:, DH:])


def _mm2(agg1, nd, b1, W2, ns):
    return pl.pallas_call(
        _mm2_body,
        grid=(_NB, NC),
        in_specs=[
            pl.BlockSpec((_BN, DH), lambda b, p: (b, 0)),
            pl.BlockSpec((_BN, DH), lambda b, p: (b + _NB, 0)),
            pl.BlockSpec((_BN, 1), lambda b, p: (b, 0)),
            pl.BlockSpec((1, D), lambda b, p: (0, 0)),
            pl.BlockSpec((D, D), lambda b, p: (0, 0)),
            pl.BlockSpec((_BN, 1), lambda b, p: (b, 0)),
        ],
        out_specs=pl.BlockSpec((_BN, DH), lambda b, p: (p * _NB + b, 0)),
        out_shape=jax.ShapeDtypeStruct((NC * N2, DH), jnp.float32),
    )(agg1, agg1, nd, b1, W2, ns)


def _final_body(a0_ref, a1_ref, nd_ref, b2_ref, out_ref):
    a = jnp.concatenate([a0_ref[...], a1_ref[...]], axis=1)
    out_ref[...] = a * nd_ref[...] + b2_ref[...]


def _final(agg2, nd, b2):
    return pl.pallas_call(
        _final_body,
        grid=(_NB,),
        in_specs=[
            pl.BlockSpec((_BN, DH), lambda b: (b, 0)),
            pl.BlockSpec((_BN, DH), lambda b: (b + _NB, 0)),
            pl.BlockSpec((_BN, 1), lambda b: (b, 0)),
            pl.BlockSpec((1, D), lambda b: (0, 0)),
        ],
        out_specs=pl.BlockSpec((_BN, D), lambda b: (b, 0)),
        out_shape=jax.ShapeDtypeStruct((N, D), jnp.float32),
    )(agg2, agg2, nd, b2)


@jax.jit
def kernel(x, edge_index, W1, b1, W2, b2):
    pad = jnp.full((NCHP * CH - E,), N, jnp.int32)
    srcd = jnp.concatenate([edge_index[0], pad]).reshape(NCHP, CH)
    dstd = jnp.concatenate([edge_index[1], pad]).reshape(NCHP, CH)

    degp = _deg(srcd, dstd).reshape(NC * NS, 2, N2)
    ns, nd = _norms(degp)

    xp = jnp.pad(x, ((0, N2 - N), (0, 0)))
    zrows = jnp.zeros((ROWS_PER_TILE, DH), jnp.float32)

    hs1 = _mm1(xp, W1, ns)
    agg1 = _agg(hs1, srcd, dstd, zrows)

    hs2 = _mm2(agg1, nd, b1.reshape(1, D), W2, ns)
    agg2 = _agg(hs2, srcd, dstd, zrows)

    return _final(agg2, nd, b2.reshape(1, D))


# speedup vs baseline: 1.4496x; 1.0199x over previous
import jax
import jax.numpy as jnp
from jax import lax
from jax.experimental import pallas as pl
from jax.experimental.pallas import tpu as pltpu
from jax.experimental.pallas import tpu_sc as plsc

N = 10000
E = 160000
D = 256
DH = 128
N2 = 10240
NC = 2
NS = 16
NW = NC * NS
CH = 128
NCHUNK = E // CH
NCHP = 1280
CPT = NCHP // NS
CPW = NCHP // NW
ROWS_PER_TILE = N2 // NS

_mesh = plsc.VectorSubcoreMesh(
    core_axis_name="c", subcore_axis_name="s", num_cores=NC, num_subcores=NS
)
_sc_params = pltpu.CompilerParams(needs_layout_passes=False)


def _deg_body(src_hbm, dst_hbm, out_hbm, srcall, dstall, dego, degi):
    c = lax.axis_index("c")
    s = lax.axis_index("s")
    wid = s * NC + c
    zeros16 = jnp.zeros((16,), jnp.float32)
    ones16 = jnp.ones((16,), jnp.float32)

    pltpu.sync_copy(src_hbm.at[pl.ds(wid * CPW, CPW)], srcall)
    pltpu.sync_copy(dst_hbm.at[pl.ds(wid * CPW, CPW)], dstall)

    def zero_body(i, _):
        dego[pl.ds(i * 16, 16)] = zeros16
        degi[pl.ds(i * 16, 16)] = zeros16
        return 0

    lax.fori_loop(0, N2 // 16, zero_body, 0)

    def chunk_body(i, _):
        for j in range(CH // 16):
            si = srcall[i, pl.ds(16 * j, 16)]
            plsc.addupdate_scatter(dego, [si], ones16)
            di = dstall[i, pl.ds(16 * j, 16)]
            plsc.addupdate_scatter(degi, [di], ones16)
        return 0

    lax.fori_loop(0, CPW, chunk_body, 0)
    pltpu.sync_copy(dego, out_hbm.at[c, s, 0])
    pltpu.sync_copy(degi, out_hbm.at[c, s, 1])


_deg = pl.kernel(
    _deg_body,
    out_type=jax.ShapeDtypeStruct((NC, NS, 2, N2), jnp.float32),
    mesh=_mesh,
    scratch_types=[
        pltpu.VMEM((CPW, CH), jnp.int32),
        pltpu.VMEM((CPW, CH), jnp.int32),
        pltpu.VMEM((N2,), jnp.float32),
        pltpu.VMEM((N2,), jnp.float32),
    ],
    compiler_params=_sc_params,
)


def _agg_body(hs_hbm, src_hbm, dst_hbm, zrows_hbm, out_hbm,
              srcb0, dstb0, rows0, agg_sh, g0):
    c = lax.axis_index("c")
    s = lax.axis_index("s")
    off = c * N2

    pltpu.sync_copy(zrows_hbm, agg_sh.at[pl.ds(s * ROWS_PER_TILE,
                                               ROWS_PER_TILE)])
    plsc.subcore_barrier()

    def adjust(buf):
        for j in range(CH // 16):
            sl = pl.ds(16 * j, 16)
            buf[sl] = buf[sl] + off

    nch = NCHUNK // NS + jnp.where(s < NCHUNK - (NCHUNK // NS) * NS, 1, 0)
    cbase = s * (NCHUNK // NS) + jnp.minimum(s, NCHUNK - (NCHUNK // NS) * NS)

    def chunk_body(i, _):
        ch = cbase + i
        pltpu.sync_copy(src_hbm.at[ch], srcb0)
        pltpu.sync_copy(dst_hbm.at[ch], dstb0)
        adjust(srcb0)
        pltpu.async_copy(hs_hbm.at[srcb0], rows0, g0).wait()
        pltpu.sync_copy(rows0, agg_sh.at[dstb0], add=True)
        return 0

    lax.fori_loop(0, nch, chunk_body, 0)
    plsc.subcore_barrier()
    pltpu.sync_copy(
        agg_sh.at[pl.ds(s * ROWS_PER_TILE, ROWS_PER_TILE)],
        out_hbm.at[pl.ds(c * N2 + s * ROWS_PER_TILE, ROWS_PER_TILE)],
    )


_agg = pl.kernel(
    _agg_body,
    out_type=jax.ShapeDtypeStruct((NC * N2, DH), jnp.float32),
    mesh=_mesh,
    scratch_types=[
        pltpu.VMEM((CH,), jnp.int32),
        pltpu.VMEM((CH,), jnp.int32),
        pltpu.VMEM((CH, DH), jnp.float32),
        pltpu.VMEM_SHARED((N2, DH), jnp.float32),
        pltpu.SemaphoreType.DMA,
    ],
    compiler_params=_sc_params,
)


_HI = jax.lax.Precision.HIGHEST
_BN = 1024
_NB = N2 // _BN
_NORM_BN = 256


def _mm1_body(degp_ref, x_ref, w_ref, out_ref, ns_ref, nd_ref, ns_sc, nd_sc):
    b = pl.program_id(0)
    p = pl.program_id(1)

    @pl.when((b == 0) & (p == 0))
    def _():
        ii = lax.broadcasted_iota(jnp.int32, (_NORM_BN, _NORM_BN), 0)
        jj = lax.broadcasted_iota(jnp.int32, (_NORM_BN, _NORM_BN), 1)
        ident = jnp.where(ii == jj, 1.0, 0.0)

        def nbody(i, _):
            sl = pl.ds(i * _NORM_BN, _NORM_BN)
            d = jnp.sum(degp_ref[:, :, sl], axis=0)
            cols = lax.dot_general(ident, d, (((1,), (1,)), ((), ())),
                                   precision=_HI)
            do = cols[:, 0:1]
            di = cols[:, 1:2]
            ns_sc[sl, :] = jnp.where(
                do > 0.0, lax.rsqrt(jnp.maximum(do, 1e-12)), 0.0)
            nd_sc[sl, :] = jnp.where(
                di > 0.0, lax.rsqrt(jnp.maximum(di, 1e-12)), 0.0)
            return 0

        lax.fori_loop(0, N2 // _NORM_BN, nbody, 0)

    nsb = ns_sc[pl.ds(b * _BN, _BN), :]
    h = jnp.dot(x_ref[...], w_ref[...], precision=_HI)
    hs = h * nsb
    out_ref[...] = jnp.where(p == 0, hs[:, :DH], hs[:, DH:])
    ns_ref[...] = nsb
    nd_ref[...] = nd_sc[pl.ds(b * _BN, _BN), :]


def _mm1(degp, x, W1):
    return pl.pallas_call(
        _mm1_body,
        grid=(_NB, NC),
        in_specs=[
            pl.BlockSpec((NW, 2, N2), lambda b, p: (0, 0, 0)),
            pl.BlockSpec((_BN, D), lambda b, p: (b, 0)),
            pl.BlockSpec((D, D), lambda b, p: (0, 0)),
        ],
        out_specs=[
            pl.BlockSpec((_BN, DH), lambda b, p: (p * _NB + b, 0)),
            pl.BlockSpec((_BN, 1), lambda b, p: (b, 0)),
            pl.BlockSpec((_BN, 1), lambda b, p: (b, 0)),
        ],
        out_shape=[
            jax.ShapeDtypeStruct((NC * N2, DH), jnp.float32),
            jax.ShapeDtypeStruct((N2, 1), jnp.float32),
            jax.ShapeDtypeStruct((N2, 1), jnp.float32),
        ],
        scratch_shapes=[
            pltpu.VMEM((N2, 1), jnp.float32),
            pltpu.VMEM((N2, 1), jnp.float32),
        ],
    )(degp, x, W1)


def _mm2_body(a0_ref, a1_ref, nd_ref, b1_ref, w_ref, ns_ref, out_ref):
    p = pl.program_id(1)
    a = jnp.concatenate([a0_ref[...], a1_ref[...]], axis=1)
    t = jnp.maximum(a * nd_ref[...] + b1_ref[...], 0.0)
    h = jnp.dot(t, w_ref[...], precision=_HI)
    hs = h * ns_ref[...]
    out_ref[...] = jnp.where(p == 0, hs[:, :DH], hs[:, DH:])


def _mm2(agg1, nd, b1, W2, ns):
    return pl.pallas_call(
        _mm2_body,
        grid=(_NB, NC),
        in_specs=[
            pl.BlockSpec((_BN, DH), lambda b, p: (b, 0)),
            pl.BlockSpec((_BN, DH), lambda b, p: (b + _NB, 0)),
            pl.BlockSpec((_BN, 1), lambda b, p: (b, 0)),
            pl.BlockSpec((1, D), lambda b, p: (0, 0)),
            pl.BlockSpec((D, D), lambda b, p: (0, 0)),
            pl.BlockSpec((_BN, 1), lambda b, p: (b, 0)),
        ],
        out_specs=pl.BlockSpec((_BN, DH), lambda b, p: (p * _NB + b, 0)),
        out_shape=jax.ShapeDtypeStruct((NC * N2, DH), jnp.float32),
    )(agg1, agg1, nd, b1, W2, ns)


def _final_body(a0_ref, a1_ref, nd_ref, b2_ref, out_ref):
    a = jnp.concatenate([a0_ref[...], a1_ref[...]], axis=1)
    out_ref[...] = a * nd_ref[...] + b2_ref[...]


def _final(agg2, nd, b2):
    return pl.pallas_call(
        _final_body,
        grid=(_NB,),
        in_specs=[
            pl.BlockSpec((_BN, DH), lambda b: (b, 0)),
            pl.BlockSpec((_BN, DH), lambda b: (b + _NB, 0)),
            pl.BlockSpec((_BN, 1), lambda b: (b, 0)),
            pl.BlockSpec((1, D), lambda b: (0, 0)),
        ],
        out_specs=pl.BlockSpec((_BN, D), lambda b: (b, 0)),
        out_shape=jax.ShapeDtypeStruct((N, D), jnp.float32),
    )(agg2, agg2, nd, b2)


@jax.jit
def kernel(x, edge_index, W1, b1, W2, b2):
    pad = jnp.full((NCHP * CH - E,), N, jnp.int32)
    srcd = jnp.concatenate([edge_index[0], pad]).reshape(NCHP, CH)
    dstd = jnp.concatenate([edge_index[1], pad]).reshape(NCHP, CH)

    degp = _deg(srcd, dstd).reshape(NC * NS, 2, N2)
    zrows = jnp.zeros((ROWS_PER_TILE, DH), jnp.float32)

    hs1, ns, nd = _mm1(degp, x, W1)
    agg1 = _agg(hs1, srcd, dstd, zrows)

    hs2 = _mm2(agg1, nd, b1.reshape(1, D), W2, ns)
    agg2 = _agg(hs2, srcd, dstd, zrows)

    return _final(agg2, nd, b2.reshape(1, D))
